# baseline (speedup 1.0000x reference)
import functools

import jax
import jax.numpy as jnp
from jax import lax
from jax.experimental import pallas as pl
from jax.experimental.pallas import tpu as pltpu
from jax.experimental.pallas import tpu_sc as plsc

VOCAB = 1000000
EMBED_DIM = 32
SEQ = 200
BATCH = 4096

_INFO = plsc.get_sparse_core_info()
_NC, _NS, _L = _INFO.num_cores, _INFO.num_subcores, _INFO.num_lanes
_NW = _NC * _NS
_BBLK = BATCH // _NW
_TG = 8


def _emb_kernel(xT_hbm, table4_hbm, out_hbm,
                x_v, q_v, off_v, g_v, o_v, gsem, osem0, osem1):
    wid = lax.axis_index("s") * _NC + lax.axis_index("c")
    col0 = wid * _BBLK
    osems = (osem0, osem1)
    tok16 = lax.iota(jnp.int32, _L)

    def do_chunk(t, j, s):
        pltpu.async_copy(table4_hbm.at[q_v.at[j]], g_v.at[s], gsem).wait()
        for tb in range(_BBLK // _L):
            colbase = off_v[j, pl.ds(tb * _L, _L)] * 32
            rows = tok16 + (tb * _L)
            for d in range(EMBED_DIM):
                vals = plsc.load_gather(g_v.at[s], [rows, colbase + d])
                o_v[s, d, pl.ds(tb * _L, _L)] = vals
        pltpu.async_copy(
            o_v.at[s], out_hbm.at[t, :, pl.ds(col0, _BBLK)], osems[s]
        )

    def wait_out(s):
        pltpu.make_async_copy(
            o_v.at[s], out_hbm.at[0, :, pl.ds(col0, _BBLK)], osems[s]
        ).wait()

    def stage(i, _):
        pltpu.sync_copy(xT_hbm.at[pl.ds(i * _TG, _TG), pl.ds(col0, _BBLK)], x_v)
        for j in range(_TG):
            for tb in range(_BBLK // _L):
                v = x_v[j, pl.ds(tb * _L, _L)]
                q_v[j, pl.ds(tb * _L, _L)] = lax.shift_right_logical(v, 2)
                off_v[j, pl.ds(tb * _L, _L)] = lax.bitwise_and(v, 3)
        for j in range(_TG):
            @pl.when(jnp.logical_or(i > 0, j >= 2))
            def _():
                wait_out(j % 2)
            do_chunk(i * _TG + j, j, j % 2)
        return 0

    lax.fori_loop(0, SEQ // _TG, stage, 0)
    wait_out(0)
    wait_out(1)


def kernel(x, table):
    assert x.shape == (BATCH, SEQ) and table.shape == (VOCAB, EMBED_DIM)
    xT = x.T.astype(jnp.int32)
    table4 = table.reshape(VOCAB // 4, 4 * EMBED_DIM)

    k = functools.partial(
        pl.kernel,
        mesh=plsc.VectorSubcoreMesh(core_axis_name="c", subcore_axis_name="s"),
        out_type=jax.ShapeDtypeStruct((SEQ, EMBED_DIM, BATCH), jnp.float32),
        scratch_types=[
            pltpu.VMEM((_TG, _BBLK), jnp.int32),
            pltpu.VMEM((_TG, _BBLK), jnp.int32),
            pltpu.VMEM((_TG, _BBLK), jnp.int32),
            pltpu.VMEM((2, _BBLK, 128), jnp.float32),
            pltpu.VMEM((2, EMBED_DIM, _BBLK), jnp.float32),
            pltpu.SemaphoreType.DMA,
            pltpu.SemaphoreType.DMA,
            pltpu.SemaphoreType.DMA,
        ],
        compiler_params=pltpu.CompilerParams(use_tc_tiling_on_sc=True,
                                              needs_layout_passes=False),
    )(_emb_kernel)

    out = k(xT, table4)
    return out.transpose(2, 0, 1)


# speedup vs baseline: 1.2320x; 1.2320x over previous
import functools

import jax
import jax.numpy as jnp
from jax import lax
from jax.experimental import pallas as pl
from jax.experimental.pallas import tpu as pltpu
from jax.experimental.pallas import tpu_sc as plsc

VOCAB = 1000000
EMBED_DIM = 32
SEQ = 200
BATCH = 4096

_INFO = plsc.get_sparse_core_info()
_NC, _NS, _L = _INFO.num_cores, _INFO.num_subcores, _INFO.num_lanes
_NW = _NC * _NS
_BBLK = BATCH // _NW
_NG = 4


def _emb_kernel(qT_hbm, cbT_hbm, table4_hbm, out_hbm,
                q_v, cb_v, g_v, o_v,
                gsem0, gsem1, gsem2, gsem3, osem0, osem1):
    wid = lax.axis_index("s") * _NC + lax.axis_index("c")
    col0 = wid * _BBLK
    gsems = (gsem0, gsem1, gsem2, gsem3)
    osems = (osem0, osem1)
    tok16 = lax.iota(jnp.int32, _L)

    pltpu.sync_copy(qT_hbm.at[:, pl.ds(col0, _BBLK)], q_v)
    pltpu.sync_copy(cbT_hbm.at[:, pl.ds(col0, _BBLK)], cb_v)

    def fire_gather(t, s):
        pltpu.async_copy(table4_hbm.at[q_v.at[t]], g_v.at[s], gsems[s])

    def wait_gather(s):
        pltpu.make_async_copy(
            table4_hbm.at[q_v.at[0]], g_v.at[s], gsems[s]).wait()

    def wait_out(s):
        pltpu.make_async_copy(
            o_v.at[s], out_hbm.at[0, :, pl.ds(col0, _BBLK)], osems[s]
        ).wait()

    for t in range(_NG - 1):
        fire_gather(t, t)

    def body(i, _):
        for b in range(_NG):
            t = i * _NG + b
            wait_gather(b)
            @pl.when(t + _NG - 1 < SEQ)
            def _():
                fire_gather(t + _NG - 1, (b + _NG - 1) % _NG)
            so = b % 2
            @pl.when(t >= 2)
            def _():
                wait_out(so)
            for tb in range(_BBLK // _L):
                cb = cb_v[t, pl.ds(tb * _L, _L)]
                rows = tok16 + (tb * _L)
                for d in range(EMBED_DIM):
                    o_v[so, d, pl.ds(tb * _L, _L)] = plsc.load_gather(
                        g_v.at[b], [rows, cb + d])
            pltpu.async_copy(
                o_v.at[so], out_hbm.at[t, :, pl.ds(col0, _BBLK)], osems[so]
            )
        return 0

    lax.fori_loop(0, SEQ // _NG, body, 0)
    wait_out(0)
    wait_out(1)


def kernel(x, table):
    assert x.shape == (BATCH, SEQ) and table.shape == (VOCAB, EMBED_DIM)
    xi = x.astype(jnp.int32)
    qT = (xi >> 2).T
    cbT = ((xi & 3) * EMBED_DIM).T
    table4 = table.reshape(VOCAB // 4, 4 * EMBED_DIM)

    k = functools.partial(
        pl.kernel,
        mesh=plsc.VectorSubcoreMesh(core_axis_name="c", subcore_axis_name="s"),
        out_type=jax.ShapeDtypeStruct((SEQ, EMBED_DIM, BATCH), jnp.float32),
        scratch_types=[
            pltpu.VMEM((SEQ, _BBLK), jnp.int32),
            pltpu.VMEM((SEQ, _BBLK), jnp.int32),
            pltpu.VMEM((_NG, _BBLK, 128), jnp.float32),
            pltpu.VMEM((2, EMBED_DIM, _BBLK), jnp.float32),
            pltpu.SemaphoreType.DMA,
            pltpu.SemaphoreType.DMA,
            pltpu.SemaphoreType.DMA,
            pltpu.SemaphoreType.DMA,
            pltpu.SemaphoreType.DMA,
            pltpu.SemaphoreType.DMA,
        ],
        compiler_params=pltpu.CompilerParams(use_tc_tiling_on_sc=True,
                                             needs_layout_passes=False),
    )(_emb_kernel)

    out = k(qT, cbT, table4)
    return out.transpose(2, 0, 1)
